# baseline (speedup 1.0000x reference)
import functools

import jax
import jax.numpy as jnp
from jax import lax
from jax.experimental import pallas as pl
from jax.experimental.pallas import tpu as pltpu
from jax.experimental.pallas import tpu_sc as plsc

B = 16384
D = 64
H1 = 256
H2 = 128
NC = 2
NS = 16
NW = NC * NS
BPW = B // NW
CH = 128
NCH = BPW // CH

MIN_RATING = 0.5
MAX_RATING = 5.0


def _sc_gather_body(uid_hbm, mid_hbm, ut_hbm, mt_hbm, uo_hbm, mo_hbm,
                    uidx_v, midx_v, urows_v, mrows_v, usem, msem):
    wid = lax.axis_index("s") * NC + lax.axis_index("c")
    base = wid * BPW
    pltpu.sync_copy(uid_hbm.at[wid], uidx_v)
    pltpu.sync_copy(mid_hbm.at[wid], midx_v)
    copies = []
    for j in range(NCH):
        copies.append(pltpu.async_copy(
            ut_hbm.at[uidx_v.at[j]], urows_v.at[pl.ds(j * CH, CH)], usem))
        copies.append(pltpu.async_copy(
            mt_hbm.at[midx_v.at[j]], mrows_v.at[pl.ds(j * CH, CH)], msem))
    for c in copies:
        c.wait()
    pltpu.sync_copy(urows_v, uo_hbm.at[pl.ds(base, BPW)])
    pltpu.sync_copy(mrows_v, mo_hbm.at[pl.ds(base, BPW)])


def _sc_gather(uid, mid, user_table, movie_table):
    mesh = plsc.VectorSubcoreMesh(
        core_axis_name="c", subcore_axis_name="s",
        num_cores=NC, num_subcores=NS)
    f = pl.kernel(
        _sc_gather_body,
        out_type=(jax.ShapeDtypeStruct((B, D), jnp.float32),
                  jax.ShapeDtypeStruct((B, D), jnp.float32)),
        mesh=mesh,
        scratch_types=[
            pltpu.VMEM((NCH, CH), jnp.int32),
            pltpu.VMEM((NCH, CH), jnp.int32),
            pltpu.VMEM((BPW, D), jnp.float32),
            pltpu.VMEM((BPW, D), jnp.float32),
            pltpu.SemaphoreType.DMA,
            pltpu.SemaphoreType.DMA,
        ],
        compiler_params=pltpu.CompilerParams(use_tc_tiling_on_sc=False),
    )
    return f(uid, mid, user_table, movie_table)


BB = 2048


def _mlp_body(w1u_ref, w1m_ref, b1_ref, w2_ref, b2_ref, w3_ref, b3_ref,
              u_ref, m_ref, out_ref):
    h = jnp.dot(u_ref[...], w1u_ref[...], preferred_element_type=jnp.float32,
                precision=lax.Precision.HIGHEST)
    h = h + jnp.dot(m_ref[...], w1m_ref[...],
                    preferred_element_type=jnp.float32,
                    precision=lax.Precision.HIGHEST)
    h = jnp.maximum(h + b1_ref[...], 0.0)
    h = jnp.dot(h, w2_ref[...], preferred_element_type=jnp.float32,
                precision=lax.Precision.HIGHEST)
    h = jnp.maximum(h + b2_ref[...], 0.0)
    o = jnp.sum(h * w3_ref[...], axis=1, keepdims=True) + b3_ref[...]
    out_ref[...] = MIN_RATING + (MAX_RATING - MIN_RATING) * jax.nn.sigmoid(o)


def _mlp(u_emb, m_emb, w1u, w1m, b1, w2, b2, w3, b3):
    grid = B // BB
    wspec = lambda shape: pl.BlockSpec(shape, lambda i: (0, 0))
    return pl.pallas_call(
        _mlp_body,
        grid=(grid,),
        in_specs=[
            wspec((D, H1)), wspec((D, H1)), wspec((1, H1)),
            wspec((H1, H2)), wspec((1, H2)), wspec((1, H2)), wspec((1, 1)),
            pl.BlockSpec((BB, D), lambda i: (i, 0)),
            pl.BlockSpec((BB, D), lambda i: (i, 0)),
        ],
        out_specs=pl.BlockSpec((BB, 1), lambda i: (i, 0)),
        out_shape=jax.ShapeDtypeStruct((B, 1), jnp.float32),
    )(w1u, w1m, b1, w2, b2, w3, b3, u_emb, m_emb)


def kernel(user_ids, movie_ids, user_table, movie_table, W1, b1, W2, b2, W3, b3):
    uid = user_ids.astype(jnp.int32).reshape(NW, NCH, CH)
    mid = movie_ids.astype(jnp.int32).reshape(NW, NCH, CH)
    u_emb, m_emb = _sc_gather(uid, mid, user_table, movie_table)
    w1u = W1[:, :D].T
    w1m = W1[:, D:].T
    out = _mlp(u_emb, m_emb, w1u, w1m, b1.reshape(1, H1),
               W2.T, b2.reshape(1, H2), W3, b3.reshape(1, 1))
    return out.reshape(B)
